# Initial kernel scaffold; baseline (speedup 1.0000x reference)
import jax
import jax.numpy as jnp
from jax.experimental import pallas as pl


def kernel(user_emb_src, user_emb_tgt, item_emb_src, item_emb_tgt, edge_u_src, edge_i_src, edge_u_tgt, edge_i_tgt):
    raise NotImplementedError("write your pallas kernel here")


# speedup vs baseline: 10.0596x; 10.0596x over previous
import functools

import jax
import jax.numpy as jnp
from jax import lax
from jax.experimental import pallas as pl
from jax.experimental.pallas import tpu as pltpu
from jax.experimental.pallas import tpu_sc as plsc

NU = 100000
NI = 60000
E = 600000
DUM = 64
NUP = 100096
LPT = NUP // 16
BLK_E = 512
DBLK = 4
NBLK = 74
EPAD = 16 * NBLK * BLK_E
ER = EPAD // 128
QROW = NBLK * DBLK
BLK_R = 640
NFULL = LPT // BLK_R
RTAIL = LPT - NFULL * BLK_R
ZR = 128
SLOT_U, SLOT_H, SLOT_V = 0, 4, 12
NSLOT = 20

_f32 = jnp.float32


def _fill(buf, nrows, val):
    v = jnp.full((16,), val, _f32)

    def body(i, _):
        for u in range(4):
            buf[i * 4 + u, :] = v
        return _

    lax.fori_loop(0, nrows // 4, body, None)


def _sc_body(edges, ue, big, inv, row0, row1, idxga, idxsa, idxgb, idxsb,
             zbuf, obuf, acc, gsa, gsb, ssa, ssb, zsem):
    c = lax.axis_index("c")
    s = lax.axis_index("s")
    barrier = plsc.subcore_barrier

    _fill(zbuf, ZR, 0.0)
    _fill(obuf, ZR, 1.0)

    def zero_acc():
        base = s * LPT

        def grp(g, _):
            off = g * (8 * ZR)
            descs = [pltpu.async_copy(zbuf,
                                      acc.at[pl.ds(base + off + j * ZR, ZR)],
                                      zsem)
                     for j in range(8)]
            for d_ in descs:
                d_.wait()
            return _

        lax.fori_loop(0, LPT // (8 * ZR), grp, None)
        tail = LPT - (LPT // (8 * ZR)) * 8 * ZR
        pltpu.async_copy(zbuf.at[pl.ds(0, tail)],
                         acc.at[pl.ds(base + LPT - tail, tail)], zsem).wait()

    def count_sum(sdim):
        def cbody(i, _):
            for t, (idxs, sem) in enumerate(((idxsa, ssa), (idxsb, ssb))):
                rbase = s * QROW + (2 * i + t) * DBLK
                pltpu.sync_copy(edges.at[sdim].at[pl.ds(rbase, DBLK)], idxs)
                for j in range(DBLK):
                    pltpu.async_copy(obuf, acc.at[idxs.at[j]], sem, add=True)
            for idxs, sem in ((idxsa, ssa), (idxsb, ssb)):
                for j in range(DBLK):
                    pltpu.make_async_copy(obuf, acc.at[idxs.at[j]],
                                          sem).wait()
            return _

        lax.fori_loop(0, NBLK // 2, cbody, None)

    def seg_sum(src, gdim, sdim):
        def g_fire(idxg, idxs, buf, sem, b):
            rbase = s * QROW + b * DBLK
            pltpu.sync_copy(edges.at[gdim].at[pl.ds(rbase, DBLK)], idxg)
            pltpu.sync_copy(edges.at[sdim].at[pl.ds(rbase, DBLK)], idxs)
            for j in range(DBLK):
                pltpu.async_copy(src.at[idxg.at[j]],
                                 buf.at[pl.ds(j * 128, 128)], sem)

        def g_drain(idxg, buf, sem):
            for j in range(DBLK):
                pltpu.make_async_copy(src.at[idxg.at[j]],
                                      buf.at[pl.ds(j * 128, 128)], sem).wait()

        def s_fire(idxs, buf, sem):
            for j in range(DBLK):
                pltpu.async_copy(buf.at[pl.ds(j * 128, 128)],
                                 acc.at[idxs.at[j]], sem, add=True)

        def s_drain(idxs, buf, sem):
            for j in range(DBLK):
                pltpu.make_async_copy(buf.at[pl.ds(j * 128, 128)],
                                      acc.at[idxs.at[j]], sem).wait()

        g_fire(idxga, idxsa, row0, gsa, jnp.int32(0))

        def body(i, _):
            g_drain(idxga, row0, gsa)
            s_fire(idxsa, row0, ssa)
            g_fire(idxgb, idxsb, row1, gsb, 2 * i + 1)
            g_drain(idxgb, row1, gsb)
            s_drain(idxsa, row0, ssa)
            s_fire(idxsb, row1, ssb)

            @pl.when(i + 1 < NBLK // 2)
            def _fire_next():
                g_fire(idxga, idxsa, row0, gsa, 2 * i + 2)

            s_drain(idxsb, row1, ssb)
            return _

        lax.fori_loop(0, NBLK // 2, body, None)

    def dump(out, inv_base=None, make_inv=False):
        rbase = s * LPT

        def block(off, n):
            pltpu.sync_copy(acc.at[pl.ds(rbase + off, n)],
                            row0.at[pl.ds(0, n)])
            if inv_base is not None:
                pltpu.sync_copy(
                    inv.at[c].at[pl.ds(inv_base + rbase + off, n)],
                    row1.at[pl.ds(0, n)])

            def body(i, _):
                for u in range(4):
                    r = i * 4 + u
                    x = row0[r, :]
                    if make_inv:
                        row0[r, :] = 1.0 / jnp.maximum(x, 1.0)
                    else:
                        row0[r, :] = x * row1[r, :]
                return _

            lax.fori_loop(0, n // 4, body, None)
            pltpu.sync_copy(row0.at[pl.ds(0, n)],
                            out.at[pl.ds(rbase + off, n)])

        lax.fori_loop(0, NFULL, lambda t, _: (block(t * BLK_R, BLK_R), _)[1],
                      None)
        block(NFULL * BLK_R, RTAIL)

    def max_rows(a, b, out):
        rbase = s * LPT

        def block(off, n):
            pltpu.sync_copy(a.at[pl.ds(rbase + off, n)], row0.at[pl.ds(0, n)])
            pltpu.sync_copy(b.at[pl.ds(rbase + off, n)], row1.at[pl.ds(0, n)])

            def body(i, _):
                for u in range(4):
                    r = i * 4 + u
                    row0[r, :] = jnp.maximum(row0[r, :], row1[r, :])
                return _

            lax.fori_loop(0, n // 4, body, None)
            pltpu.sync_copy(row0.at[pl.ds(0, n)],
                            out.at[pl.ds(rbase + off, n)])

        lax.fori_loop(0, NFULL, lambda t, _: (block(t * BLK_R, BLK_R), _)[1],
                      None)
        block(NFULL * BLK_R, RTAIL)

    def phase_a(kk, _):
        k = 2 * c + kk
        max_rows(ue.at[0].at[k], ue.at[1].at[k], big.at[SLOT_U + k])
        return _

    lax.fori_loop(0, 2, phase_a, None)
    barrier()

    def phase_b(t, _):
        sdim = jnp.where(t < 2, 2 * t + 1, 2 * t - 4)
        zero_acc()
        barrier()
        count_sum(sdim)
        barrier()
        dump(inv.at[c].at[pl.ds(t * NUP, NUP)], make_inv=True)
        barrier()
        return _

    lax.fori_loop(0, 4, phase_b, None)

    def phase_c(kk, _):
        k = 2 * c + kk

        def layer(_l, carry):
            def one_pass(p, cc):
                d = p // 2
                u2i = (p % 2) == 0
                src = jnp.where(u2i, SLOT_U + k, SLOT_H + d * 4 + k)
                dst = jnp.where(u2i, SLOT_H + d * 4 + k, SLOT_V + d * 4 + k)
                gdim = jnp.where(u2i, 2 * d, 2 * d + 1)
                sdim = jnp.where(u2i, 2 * d + 1, 2 * d)
                inv_base = jnp.where(u2i, d, 2 + d) * NUP
                zero_acc()
                barrier()
                seg_sum(big.at[src], gdim, sdim)
                barrier()
                dump(big.at[dst], inv_base=inv_base)
                barrier()
                return cc

            lax.fori_loop(0, 4, one_pass, None)
            max_rows(big.at[SLOT_V + k], big.at[SLOT_V + 4 + k],
                     big.at[SLOT_U + k])
            barrier()
            return carry

        lax.fori_loop(0, 2, layer, None)
        return _

    lax.fori_loop(0, 2, phase_c, None)


@functools.cache
def _build():
    mesh = plsc.VectorSubcoreMesh(core_axis_name="c", subcore_axis_name="s")
    return pl.kernel(
        _sc_body,
        out_type=[
            jax.ShapeDtypeStruct((NSLOT, NUP, 16), _f32),
            jax.ShapeDtypeStruct((2, 4 * NUP, 16), _f32),
        ],
        mesh=mesh,
        scratch_types=[
            pltpu.VMEM((BLK_R, 16), _f32),
            pltpu.VMEM((BLK_R, 16), _f32),
            pltpu.VMEM((DBLK, 128), jnp.int32),
            pltpu.VMEM((DBLK, 128), jnp.int32),
            pltpu.VMEM((DBLK, 128), jnp.int32),
            pltpu.VMEM((DBLK, 128), jnp.int32),
            pltpu.VMEM((ZR, 16), _f32),
            pltpu.VMEM((ZR, 16), _f32),
            pltpu.VMEM_SHARED((NUP, 16), _f32),
            pltpu.SemaphoreType.DMA,
            pltpu.SemaphoreType.DMA,
            pltpu.SemaphoreType.DMA,
            pltpu.SemaphoreType.DMA,
            pltpu.SemaphoreType.DMA,
        ],
        compiler_params=pltpu.CompilerParams(use_tc_tiling_on_sc=False),
    )


def _pad_edges(e, dummy_base):
    pad = dummy_base + (jnp.arange(EPAD - E, dtype=jnp.int32) % DUM)
    return jnp.concatenate([e, pad])


def _to_chunk_major(x):
    n = x.shape[0]
    x = x.reshape(n, 4, 16).transpose(1, 0, 2)
    return jnp.pad(x, ((0, 0), (0, NUP - n), (0, 0)))


def _from_chunk_major(x, n):
    return x[:, :n].transpose(1, 0, 2).reshape(n, 64)


def kernel(user_emb_src, user_emb_tgt, item_emb_src, item_emb_tgt,
           edge_u_src, edge_i_src, edge_u_tgt, edge_i_tgt):
    del item_emb_src, item_emb_tgt
    edges = jnp.stack([
        _pad_edges(edge_u_src, NU), _pad_edges(edge_i_src, NI),
        _pad_edges(edge_u_tgt, NU), _pad_edges(edge_i_tgt, NI),
    ]).reshape(4, ER, 128)
    ue = jnp.stack([_to_chunk_major(user_emb_src),
                    _to_chunk_major(user_emb_tgt)])
    big, _ = _build()(edges, ue)
    return jnp.concatenate([
        _from_chunk_major(big[SLOT_V:SLOT_V + 4], NU),
        _from_chunk_major(big[SLOT_V + 4:SLOT_V + 8], NU),
        _from_chunk_major(big[SLOT_H:SLOT_H + 4], NI),
        _from_chunk_major(big[SLOT_H + 4:SLOT_H + 8], NI),
    ])


# speedup vs baseline: 12.9880x; 1.2911x over previous
import functools

import jax
import jax.numpy as jnp
import numpy as np
from jax import lax
from jax.experimental import pallas as pl
from jax.experimental.pallas import tpu as pltpu
from jax.experimental.pallas import tpu_sc as plsc

NU = 100000
NI = 60000
E = 600000
DUM = 64
NUP = 100096
LPT = NUP // 16
CLAMP = NU - LPT
BLK_E = 512
DBLK = 4
NBLK = 74
EPAD = 16 * NBLK * BLK_E
ER = EPAD // 128
QROW = NBLK * DBLK
BLK_R = 640
NFULL = LPT // BLK_R
RTAIL = LPT - NFULL * BLK_R
ZR = 128
SLOT_U, SLOT_H = 0, 4
NSLOT = 12
OUT_VS, OUT_VT, OUT_HS, OUT_HT = 0, 1, 2, 3

_f32 = jnp.float32


def _fill(buf, nrows, val):
    v = jnp.full((16,), val, _f32)

    def body(i, _):
        for u in range(4):
            buf[i * 4 + u, :] = v
        return _

    lax.fori_loop(0, nrows // 4, body, None)


def _sc_body(edges, ue_s, ue_t, big, big2, inv, row0, row1, idxga, idxsa,
             idxgb, idxsb, zbuf, obuf, acc, gsa, gsb, ssa, ssb, zsem):
    c = lax.axis_index("c")
    s = lax.axis_index("s")
    barrier = plsc.subcore_barrier

    _fill(zbuf, ZR, 0.0)
    _fill(obuf, ZR, 1.0)

    def stripe_blocks(fn):
        lax.fori_loop(0, NFULL, lambda t, _: (fn(t * BLK_R, BLK_R), _)[1],
                      None)
        fn(NFULL * BLK_R, RTAIL)

    def zero_acc():
        base = s * LPT

        def grp(g, _):
            off = g * (8 * ZR)
            descs = [pltpu.async_copy(zbuf,
                                      acc.at[pl.ds(base + off + j * ZR, ZR)],
                                      zsem)
                     for j in range(8)]
            for d_ in descs:
                d_.wait()
            return _

        lax.fori_loop(0, LPT // (8 * ZR), grp, None)
        tail = LPT - (LPT // (8 * ZR)) * 8 * ZR
        pltpu.async_copy(zbuf.at[pl.ds(0, tail)],
                         acc.at[pl.ds(base + LPT - tail, tail)], zsem).wait()

    def count_sum(sdim):
        def cbody(i, _):
            for t, (idxs, sem) in enumerate(((idxsa, ssa), (idxsb, ssb))):
                rbase = s * QROW + (2 * i + t) * DBLK
                pltpu.sync_copy(edges.at[sdim].at[pl.ds(rbase, DBLK)], idxs)
                for j in range(DBLK):
                    pltpu.async_copy(obuf, acc.at[idxs.at[j]], sem, add=True)
            for idxs, sem in ((idxsa, ssa), (idxsb, ssb)):
                for j in range(DBLK):
                    pltpu.make_async_copy(obuf, acc.at[idxs.at[j]],
                                          sem).wait()
            return _

        lax.fori_loop(0, NBLK // 2, cbody, None)

    def seg_sum(src, gdim, sdim):
        def g_fire(idxg, idxs, buf, sem, b):
            rbase = s * QROW + b * DBLK
            pltpu.sync_copy(edges.at[gdim].at[pl.ds(rbase, DBLK)], idxg)
            pltpu.sync_copy(edges.at[sdim].at[pl.ds(rbase, DBLK)], idxs)
            for j in range(DBLK):
                pltpu.async_copy(src.at[idxg.at[j]],
                                 buf.at[pl.ds(j * 128, 128)], sem)

        def g_drain(idxg, buf, sem):
            for j in range(DBLK):
                pltpu.make_async_copy(src.at[idxg.at[j]],
                                      buf.at[pl.ds(j * 128, 128)], sem).wait()

        def s_fire(idxs, buf, sem):
            for j in range(DBLK):
                pltpu.async_copy(buf.at[pl.ds(j * 128, 128)],
                                 acc.at[idxs.at[j]], sem, add=True)

        def s_drain(idxs, buf, sem):
            for j in range(DBLK):
                pltpu.make_async_copy(buf.at[pl.ds(j * 128, 128)],
                                      acc.at[idxs.at[j]], sem).wait()

        g_fire(idxga, idxsa, row0, gsa, jnp.int32(0))

        def body(i, _):
            g_drain(idxga, row0, gsa)
            s_fire(idxsa, row0, ssa)
            g_fire(idxgb, idxsb, row1, gsb, 2 * i + 1)
            g_drain(idxgb, row1, gsb)
            s_drain(idxsa, row0, ssa)
            s_fire(idxsb, row1, ssb)

            @pl.when(i + 1 < NBLK // 2)
            def _fire_next():
                g_fire(idxga, idxsa, row0, gsa, 2 * i + 2)

            s_drain(idxsb, row1, ssb)
            return _

        lax.fori_loop(0, NBLK // 2, body, None)

    def dump_inv(t):
        rbase = s * LPT

        def block(off, n):
            pltpu.sync_copy(acc.at[pl.ds(rbase + off, n)],
                            row0.at[pl.ds(0, n)])

            def body(i, _):
                for u in range(4):
                    r = i * 4 + u
                    row0[r, :] = 1.0 / jnp.maximum(row0[r, :], 1.0)
                return _

            lax.fori_loop(0, n // 4, body, None)
            pltpu.sync_copy(row0.at[pl.ds(0, n)],
                            inv.at[c].at[pl.ds(t * NUP + rbase + off, n)])

        stripe_blocks(block)

    def dump_pass(u2i, hslot, out2, col, inv_base):
        rbase = s * LPT

        def block(off, n):
            pltpu.sync_copy(acc.at[pl.ds(rbase + off, n)],
                            row0.at[pl.ds(0, n)])
            pltpu.sync_copy(inv.at[c].at[pl.ds(inv_base + rbase + off, n)],
                            row1.at[pl.ds(0, n)])

            def body(i, _):
                for u in range(4):
                    r = i * 4 + u
                    row0[r, :] = row0[r, :] * row1[r, :]
                return _

            lax.fori_loop(0, n // 4, body, None)
            pltpu.sync_copy(
                row0.at[pl.ds(0, n)],
                big2.at[out2].at[pl.ds(rbase + off, n),
                                 pl.ds(col * 16, 16)])

            @pl.when(u2i)
            def _also_cm():
                pltpu.sync_copy(row0.at[pl.ds(0, n)],
                                big.at[hslot].at[pl.ds(rbase + off, n)])

        stripe_blocks(block)

    def max_cols(a2, b2, k, clamp):
        rbase = jnp.minimum(s * LPT, clamp)
        cs = pl.ds(k * 16, 16)

        def block(off, n):
            pltpu.sync_copy(a2.at[pl.ds(rbase + off, n), cs],
                            row0.at[pl.ds(0, n)])
            pltpu.sync_copy(b2.at[pl.ds(rbase + off, n), cs],
                            row1.at[pl.ds(0, n)])

            def body(i, _):
                for u in range(4):
                    r = i * 4 + u
                    row0[r, :] = jnp.maximum(row0[r, :], row1[r, :])
                return _

            lax.fori_loop(0, n // 4, body, None)
            pltpu.sync_copy(row0.at[pl.ds(0, n)],
                            big.at[SLOT_U + k].at[pl.ds(rbase + off, n)])

        stripe_blocks(block)

    def phase_a(kk, _):
        k = 2 * c + kk
        max_cols(ue_s, ue_t, k, CLAMP)

        @pl.when(s == 0)
        def _pad_fill():
            pltpu.sync_copy(zbuf.at[pl.ds(0, NUP - NU)],
                            big.at[SLOT_U + k].at[pl.ds(NU, NUP - NU)])
        return _

    lax.fori_loop(0, 2, phase_a, None)
    barrier()

    def phase_b(t, _):
        sdim = jnp.where(t < 2, 2 * t + 1, 2 * t - 4)
        zero_acc()
        barrier()
        count_sum(sdim)
        barrier()
        dump_inv(t)
        barrier()
        return _

    lax.fori_loop(0, 4, phase_b, None)

    def phase_c(kk, _):
        k = 2 * c + kk

        def layer(l, carry):
            def one_pass(p, cc):
                d = p // 2
                u2i = (p % 2) == 0
                src = jnp.where(u2i, SLOT_U + k, SLOT_H + d * 4 + k)
                gdim = jnp.where(u2i, 2 * d, 2 * d + 1)
                sdim = jnp.where(u2i, 2 * d + 1, 2 * d)
                inv_base = jnp.where(u2i, d, 2 + d) * NUP
                out2 = jnp.where(u2i, 2 + d, d)
                zero_acc()
                barrier()
                seg_sum(big.at[src], gdim, sdim)
                barrier()
                dump_pass(u2i, SLOT_H + d * 4 + k, out2, k, inv_base)
                barrier()
                return cc

            lax.fori_loop(0, 4, one_pass, None)

            @pl.when(l == 0)
            def _update_u():
                max_cols(big2.at[OUT_VS], big2.at[OUT_VT], k,
                         NUP - LPT)
            barrier()
            return carry

        lax.fori_loop(0, 2, layer, None)
        return _

    lax.fori_loop(0, 2, phase_c, None)


@functools.cache
def _build():
    mesh = plsc.VectorSubcoreMesh(core_axis_name="c", subcore_axis_name="s")
    return pl.kernel(
        _sc_body,
        out_type=[
            jax.ShapeDtypeStruct((NSLOT, NUP, 16), _f32),
            jax.ShapeDtypeStruct((4, NUP, 64), _f32),
            jax.ShapeDtypeStruct((2, 4 * NUP, 16), _f32),
        ],
        mesh=mesh,
        scratch_types=[
            pltpu.VMEM((BLK_R, 16), _f32),
            pltpu.VMEM((BLK_R, 16), _f32),
            pltpu.VMEM((DBLK, 128), jnp.int32),
            pltpu.VMEM((DBLK, 128), jnp.int32),
            pltpu.VMEM((DBLK, 128), jnp.int32),
            pltpu.VMEM((DBLK, 128), jnp.int32),
            pltpu.VMEM((ZR, 16), _f32),
            pltpu.VMEM((ZR, 16), _f32),
            pltpu.VMEM_SHARED((NUP, 16), _f32),
            pltpu.SemaphoreType.DMA,
            pltpu.SemaphoreType.DMA,
            pltpu.SemaphoreType.DMA,
            pltpu.SemaphoreType.DMA,
            pltpu.SemaphoreType.DMA,
        ],
        compiler_params=pltpu.CompilerParams(use_tc_tiling_on_sc=False),
    )


_NPAD = EPAD - E
_PADS = [np.int32(b) + (np.arange(_NPAD, dtype=np.int32) % DUM)
         for b in (NU, NI, NU, NI)]


def kernel(user_emb_src, user_emb_tgt, item_emb_src, item_emb_tgt,
           edge_u_src, edge_i_src, edge_u_tgt, edge_i_tgt):
    del item_emb_src, item_emb_tgt
    edges = jnp.concatenate([
        edge_u_src, _PADS[0], edge_i_src, _PADS[1],
        edge_u_tgt, _PADS[2], edge_i_tgt, _PADS[3],
    ]).reshape(4, ER, 128)
    _, big2, _ = _build()(edges, user_emb_src, user_emb_tgt)
    return jnp.concatenate([
        big2[OUT_VS, :NU], big2[OUT_VT, :NU],
        big2[OUT_HS, :NI], big2[OUT_HT, :NI],
    ])


# speedup vs baseline: 19.8688x; 1.5298x over previous
import functools

import jax
import jax.numpy as jnp
import numpy as np
from jax import lax
from jax.experimental import pallas as pl
from jax.experimental.pallas import tpu as pltpu
from jax.experimental.pallas import tpu_sc as plsc

NU = 100000
NI = 60000
E = 600000
DUM = 64
NUP = 100096
LPT = NUP // 16
CLAMP = NU - LPT
DBLK = 8
NBLK = 76
NP = NBLK // 2
EPAD = 16 * NBLK * 512
ER = EPAD // 128
QROW = NBLK * 4
BLK_R = 512
NFULL = LPT // BLK_R
RTAIL = LPT - NFULL * BLK_R
ZR = 128
SLOT_U, SLOT_H = 0, 4
NSLOT = 12
OUT_VS, OUT_VT, OUT_HS, OUT_HT = 0, 1, 2, 3

_f32 = jnp.float32


def _fill(buf, nrows, val):
    v = jnp.full((16,), val, _f32)

    def body(i, _):
        for u in range(4):
            buf[i * 4 + u, :] = v
        return _

    lax.fori_loop(0, nrows // 4, body, None)


def _zero_chunks(n):
    out, off = [], 0
    while off < n:
        m = min(ZR, n - off)
        out.append((off, m))
        off += m
    return out


def _sc_body(edges, ue_s, ue_t, big, big2, inv, row0, row1, idxga, idxsa,
             idxgb, idxsb, zbuf, obuf, acc, gsa, gsb, ssa, ssb, zsem, isem):
    c = lax.axis_index("c")
    s = lax.axis_index("s")
    barrier = plsc.subcore_barrier

    _fill(zbuf, ZR, 0.0)
    _fill(obuf, ZR, 1.0)

    def stripe_blocks(fn):
        lax.fori_loop(0, NFULL, lambda t, _: (fn(t * BLK_R, BLK_R), _)[1],
                      None)
        fn(NFULL * BLK_R, RTAIL)

    def zero_acc():
        base = s * LPT

        def grp(g, _):
            off = g * (8 * ZR)
            descs = [pltpu.async_copy(zbuf,
                                      acc.at[pl.ds(base + off + j * ZR, ZR)],
                                      zsem)
                     for j in range(8)]
            for d_ in descs:
                d_.wait()
            return _

        lax.fori_loop(0, LPT // (8 * ZR), grp, None)
        tail = LPT - (LPT // (8 * ZR)) * 8 * ZR
        pltpu.async_copy(zbuf.at[pl.ds(0, tail)],
                         acc.at[pl.ds(base + LPT - tail, tail)], zsem).wait()

    def count_sum(sdim):
        def cbody(i, _):
            for t, (idxs, sem) in enumerate(((idxsa, ssa), (idxsb, ssb))):
                rbase = s * QROW + (2 * i + t) * 4
                pltpu.sync_copy(edges.at[sdim].at[pl.ds(rbase, 4)],
                                idxs.at[pl.ds(0, 4)])
                for j in range(4):
                    pltpu.async_copy(obuf, acc.at[idxs.at[j]], sem, add=True)
            for idxs, sem in ((idxsa, ssa), (idxsb, ssb)):
                for j in range(4):
                    pltpu.make_async_copy(obuf, acc.at[idxs.at[j]],
                                          sem).wait()
            return _

        lax.fori_loop(0, NBLK // 2, cbody, None)

    def seg_sum(src, gdim, sdim):
        def idx_load_sync(idxg, idxs, p):
            rbase = s * QROW + p * DBLK
            pltpu.sync_copy(edges.at[gdim].at[pl.ds(rbase, DBLK)], idxg)
            pltpu.sync_copy(edges.at[sdim].at[pl.ds(rbase, DBLK)], idxs)

        def idx_fire(idxg, idxs, p):
            rbase = s * QROW + p * DBLK
            pltpu.async_copy(edges.at[gdim].at[pl.ds(rbase, DBLK)], idxg,
                             isem)
            pltpu.async_copy(edges.at[sdim].at[pl.ds(rbase, DBLK)], idxs,
                             isem)

        def idx_drain(idxg, idxs, p):
            rbase = s * QROW + p * DBLK
            pltpu.make_async_copy(edges.at[gdim].at[pl.ds(rbase, DBLK)],
                                  idxg, isem).wait()
            pltpu.make_async_copy(edges.at[sdim].at[pl.ds(rbase, DBLK)],
                                  idxs, isem).wait()

        def g_fire(idxg, half, buf, sem):
            for j in range(4):
                pltpu.async_copy(src.at[idxg.at[half * 4 + j]],
                                 buf.at[pl.ds(j * 128, 128)], sem)

        def g_drain(idxg, half, buf, sem):
            for j in range(4):
                pltpu.make_async_copy(src.at[idxg.at[half * 4 + j]],
                                      buf.at[pl.ds(j * 128, 128)],
                                      sem).wait()

        def s_fire(idxs, half, buf, sem):
            for j in range(4):
                pltpu.async_copy(buf.at[pl.ds(j * 128, 128)],
                                 acc.at[idxs.at[half * 4 + j]], sem,
                                 add=True)

        def s_drain(idxs, half, buf, sem):
            for j in range(4):
                pltpu.make_async_copy(buf.at[pl.ds(j * 128, 128)],
                                      acc.at[idxs.at[half * 4 + j]],
                                      sem).wait()

        def stage(idxg_c, idxs_c, idxg_n, idxs_n, p):
            @pl.when(p + 1 < NP)
            def _prefetch_idx():
                idx_fire(idxg_n, idxs_n, p + 1)

            g_fire(idxg_c, 1, row1, gsb)
            g_drain(idxg_c, 0, row0, gsa)
            s_fire(idxs_c, 0, row0, ssa)
            g_drain(idxg_c, 1, row1, gsb)
            s_drain(idxs_c, 0, row0, ssa)
            s_fire(idxs_c, 1, row1, ssb)

            @pl.when(p + 1 < NP)
            def _start_next():
                idx_drain(idxg_n, idxs_n, p + 1)
                g_fire(idxg_n, 0, row0, gsa)

            s_drain(idxs_c, 1, row1, ssb)

        idx_load_sync(idxga, idxsa, jnp.int32(0))
        g_fire(idxga, 0, row0, gsa)

        def body(t, _):
            stage(idxga, idxsa, idxgb, idxsb, 2 * t)
            stage(idxgb, idxsb, idxga, idxsa, 2 * t + 1)
            return _

        lax.fori_loop(0, NP // 2, body, None)

    def mul8(n, op):
        def body(i, _):
            for u in range(8):
                op(i * 8 + u)
            return _

        lax.fori_loop(0, n // 8, body, None)

    def rezero(off, n):
        base = s * LPT + off
        return [pltpu.async_copy(zbuf.at[pl.ds(0, m)],
                                 acc.at[pl.ds(base + z, m)], zsem)
                for z, m in _zero_chunks(n)]

    def dump_inv(t):
        rbase = s * LPT

        def block(off, n):
            pltpu.sync_copy(acc.at[pl.ds(rbase + off, n)],
                            row0.at[pl.ds(0, n)])
            zd = rezero(off, n)

            def op(r):
                row0[r, :] = 1.0 / jnp.maximum(row0[r, :], 1.0)

            mul8(n, op)
            pltpu.sync_copy(row0.at[pl.ds(0, n)],
                            inv.at[c].at[pl.ds(t * NUP + rbase + off, n)])
            for d_ in zd:
                d_.wait()

        stripe_blocks(block)

    def dump_pass(u2i, w2, hslot, out2, col, inv_base):
        rbase = s * LPT

        def block(off, n):
            pltpu.sync_copy(acc.at[pl.ds(rbase + off, n)],
                            row0.at[pl.ds(0, n)])
            zd = rezero(off, n)
            pltpu.sync_copy(inv.at[c].at[pl.ds(inv_base + rbase + off, n)],
                            row1.at[pl.ds(0, n)])

            def op(r):
                row0[r, :] = row0[r, :] * row1[r, :]

            mul8(n, op)

            @pl.when(w2)
            def _w2():
                pltpu.sync_copy(
                    row0.at[pl.ds(0, n)],
                    big2.at[out2].at[pl.ds(rbase + off, n),
                                     pl.ds(col * 16, 16)])

            @pl.when(u2i)
            def _also_cm():
                pltpu.sync_copy(row0.at[pl.ds(0, n)],
                                big.at[hslot].at[pl.ds(rbase + off, n)])

            for d_ in zd:
                d_.wait()

        stripe_blocks(block)

    def max_cols(a2, b2, k, clamp):
        rbase = jnp.minimum(s * LPT, clamp)
        cs = pl.ds(k * 16, 16)

        def block(off, n):
            pltpu.sync_copy(a2.at[pl.ds(rbase + off, n), cs],
                            row0.at[pl.ds(0, n)])
            pltpu.sync_copy(b2.at[pl.ds(rbase + off, n), cs],
                            row1.at[pl.ds(0, n)])

            def op(r):
                row0[r, :] = jnp.maximum(row0[r, :], row1[r, :])

            mul8(n, op)
            pltpu.sync_copy(row0.at[pl.ds(0, n)],
                            big.at[SLOT_U + k].at[pl.ds(rbase + off, n)])

        stripe_blocks(block)

    def phase_a(kk, _):
        k = 2 * c + kk
        max_cols(ue_s, ue_t, k, CLAMP)

        @pl.when(s == 0)
        def _pad_fill():
            pltpu.sync_copy(zbuf.at[pl.ds(0, NUP - NU)],
                            big.at[SLOT_U + k].at[pl.ds(NU, NUP - NU)])
        return _

    lax.fori_loop(0, 2, phase_a, None)
    zero_acc()
    barrier()

    def phase_b(t, _):
        sdim = jnp.where(t < 2, 2 * t + 1, 2 * t - 4)
        count_sum(sdim)
        barrier()
        dump_inv(t)
        barrier()
        return _

    lax.fori_loop(0, 4, phase_b, None)

    def phase_c(kk, _):
        k = 2 * c + kk

        def layer(l, carry):
            def one_pass(p, cc):
                d = p // 2
                u2i = (p % 2) == 0
                src = jnp.where(u2i, SLOT_U + k, SLOT_H + d * 4 + k)
                gdim = jnp.where(u2i, 2 * d, 2 * d + 1)
                sdim = jnp.where(u2i, 2 * d + 1, 2 * d)
                inv_base = jnp.where(u2i, d, 2 + d) * NUP
                out2 = jnp.where(u2i, 2 + d, d)
                w2 = jnp.logical_or(jnp.logical_not(u2i), l == 1)
                seg_sum(big.at[src], gdim, sdim)
                barrier()
                dump_pass(u2i, w2, SLOT_H + d * 4 + k, out2, k, inv_base)
                barrier()
                return cc

            lax.fori_loop(0, 4, one_pass, None)

            @pl.when(l == 0)
            def _update_u():
                max_cols(big2.at[OUT_VS], big2.at[OUT_VT], k, NUP - LPT)
            barrier()
            return carry

        lax.fori_loop(0, 2, layer, None)
        return _

    lax.fori_loop(0, 2, phase_c, None)


@functools.cache
def _build():
    mesh = plsc.VectorSubcoreMesh(core_axis_name="c", subcore_axis_name="s")
    return pl.kernel(
        _sc_body,
        out_type=[
            jax.ShapeDtypeStruct((NSLOT, NUP, 16), _f32),
            jax.ShapeDtypeStruct((4, NUP, 64), _f32),
            jax.ShapeDtypeStruct((2, 4 * NUP, 16), _f32),
        ],
        mesh=mesh,
        scratch_types=[
            pltpu.VMEM((BLK_R, 16), _f32),
            pltpu.VMEM((BLK_R, 16), _f32),
            pltpu.VMEM((DBLK, 128), jnp.int32),
            pltpu.VMEM((DBLK, 128), jnp.int32),
            pltpu.VMEM((DBLK, 128), jnp.int32),
            pltpu.VMEM((DBLK, 128), jnp.int32),
            pltpu.VMEM((ZR, 16), _f32),
            pltpu.VMEM((ZR, 16), _f32),
            pltpu.VMEM_SHARED((NUP, 16), _f32),
            pltpu.SemaphoreType.DMA,
            pltpu.SemaphoreType.DMA,
            pltpu.SemaphoreType.DMA,
            pltpu.SemaphoreType.DMA,
            pltpu.SemaphoreType.DMA,
            pltpu.SemaphoreType.DMA,
        ],
        compiler_params=pltpu.CompilerParams(use_tc_tiling_on_sc=False),
    )


_NPAD = EPAD - E
_PADS = [np.int32(b) + (np.arange(_NPAD, dtype=np.int32) % DUM)
         for b in (NU, NI, NU, NI)]


def kernel(user_emb_src, user_emb_tgt, item_emb_src, item_emb_tgt,
           edge_u_src, edge_i_src, edge_u_tgt, edge_i_tgt):
    del item_emb_src, item_emb_tgt
    edges = jnp.concatenate([
        edge_u_src, _PADS[0], edge_i_src, _PADS[1],
        edge_u_tgt, _PADS[2], edge_i_tgt, _PADS[3],
    ]).reshape(4, ER, 128)
    _, big2, _ = _build()(edges, user_emb_src, user_emb_tgt)
    return jnp.concatenate([
        big2[OUT_VS, :NU], big2[OUT_VT, :NU],
        big2[OUT_HS, :NI], big2[OUT_HT, :NI],
    ])


# speedup vs baseline: 21.1603x; 1.0650x over previous
import functools

import jax
import jax.numpy as jnp
import numpy as np
from jax import lax
from jax.experimental import pallas as pl
from jax.experimental.pallas import tpu as pltpu
from jax.experimental.pallas import tpu_sc as plsc

NU = 100000
NI = 60000
E = 600000
DUM = 64
NUP = 100096
LPT = NUP // 16
CLAMP = NU - LPT
DBLK = 8
NBLK = 76
NP = NBLK // 2
EPAD = 16 * NBLK * 512
ER = EPAD // 128
QROW = NBLK * 4
BLK_R = 512
NFULL = LPT // BLK_R
RTAIL = LPT - NFULL * BLK_R
ZR = 128
SLOT_U, SLOT_H = 0, 4
NSLOT = 12
OUT_VS, OUT_VT, OUT_HS, OUT_HT = 0, 1, 2, 3

_f32 = jnp.float32


def _fill(buf, nrows, val):
    v = jnp.full((16,), val, _f32)

    def body(i, _):
        for u in range(4):
            buf[i * 4 + u, :] = v
        return _

    lax.fori_loop(0, nrows // 4, body, None)


def _zero_chunks(n):
    out, off = [], 0
    while off < n:
        m = min(ZR, n - off)
        out.append((off, m))
        off += m
    return out


def _sc_body(edges, u0, big, big2, inv, row0, row1, idxga, idxsa,
             idxgb, idxsb, zbuf, obuf, acc, gsa, gsb, ssa, ssb, zsem, isem):
    c = lax.axis_index("c")
    s = lax.axis_index("s")
    barrier = plsc.subcore_barrier

    _fill(zbuf, ZR, 0.0)
    _fill(obuf, ZR, 1.0)

    def stripe_blocks(fn):
        lax.fori_loop(0, NFULL, lambda t, _: (fn(t * BLK_R, BLK_R), _)[1],
                      None)
        fn(NFULL * BLK_R, RTAIL)

    def zero_acc():
        base = s * LPT

        def grp(g, _):
            off = g * (8 * ZR)
            descs = [pltpu.async_copy(zbuf,
                                      acc.at[pl.ds(base + off + j * ZR, ZR)],
                                      zsem)
                     for j in range(8)]
            for d_ in descs:
                d_.wait()
            return _

        lax.fori_loop(0, LPT // (8 * ZR), grp, None)
        tail = LPT - (LPT // (8 * ZR)) * 8 * ZR
        pltpu.async_copy(zbuf.at[pl.ds(0, tail)],
                         acc.at[pl.ds(base + LPT - tail, tail)], zsem).wait()

    def count_sum(sdim):
        def idx_fire(idxs, p):
            pltpu.async_copy(edges.at[sdim].at[pl.ds(s * QROW + p * DBLK,
                                                     DBLK)], idxs, isem)

        def idx_drain(idxs, p):
            pltpu.make_async_copy(edges.at[sdim].at[pl.ds(s * QROW + p * DBLK,
                                                          DBLK)], idxs,
                                  isem).wait()

        def stage(idxs_c, idxs_n, p):
            @pl.when(p + 1 < NP)
            def _prefetch():
                idx_fire(idxs_n, p + 1)
            for j in range(DBLK):
                pltpu.async_copy(obuf, acc.at[idxs_c.at[j]], ssa, add=True)
            for j in range(DBLK):
                pltpu.make_async_copy(obuf, acc.at[idxs_c.at[j]],
                                      ssa).wait()

            @pl.when(p + 1 < NP)
            def _ready():
                idx_drain(idxs_n, p + 1)

        pltpu.sync_copy(edges.at[sdim].at[pl.ds(s * QROW, DBLK)], idxsa)

        def cbody(t, _):
            stage(idxsa, idxsb, 2 * t)
            stage(idxsb, idxsa, 2 * t + 1)
            return _

        lax.fori_loop(0, NP // 2, cbody, None)

    def seg_sum(src, gdim, sdim):
        def idx_load_sync(idxg, idxs, p):
            rbase = s * QROW + p * DBLK
            pltpu.sync_copy(edges.at[gdim].at[pl.ds(rbase, DBLK)], idxg)
            pltpu.sync_copy(edges.at[sdim].at[pl.ds(rbase, DBLK)], idxs)

        def idx_fire(idxg, idxs, p):
            rbase = s * QROW + p * DBLK
            pltpu.async_copy(edges.at[gdim].at[pl.ds(rbase, DBLK)], idxg,
                             isem)
            pltpu.async_copy(edges.at[sdim].at[pl.ds(rbase, DBLK)], idxs,
                             isem)

        def idx_drain(idxg, idxs, p):
            rbase = s * QROW + p * DBLK
            pltpu.make_async_copy(edges.at[gdim].at[pl.ds(rbase, DBLK)],
                                  idxg, isem).wait()
            pltpu.make_async_copy(edges.at[sdim].at[pl.ds(rbase, DBLK)],
                                  idxs, isem).wait()

        def g_fire(idxg, half, buf, sem):
            for j in range(4):
                pltpu.async_copy(src.at[idxg.at[half * 4 + j]],
                                 buf.at[pl.ds(j * 128, 128)], sem)

        def g_drain(idxg, half, buf, sem):
            for j in range(4):
                pltpu.make_async_copy(src.at[idxg.at[half * 4 + j]],
                                      buf.at[pl.ds(j * 128, 128)],
                                      sem).wait()

        def s_fire(idxs, half, buf, sem):
            for j in range(4):
                pltpu.async_copy(buf.at[pl.ds(j * 128, 128)],
                                 acc.at[idxs.at[half * 4 + j]], sem,
                                 add=True)

        def s_drain(idxs, half, buf, sem):
            for j in range(4):
                pltpu.make_async_copy(buf.at[pl.ds(j * 128, 128)],
                                      acc.at[idxs.at[half * 4 + j]],
                                      sem).wait()

        def stage(idxg_c, idxs_c, idxg_n, idxs_n, p):
            @pl.when(p + 1 < NP)
            def _prefetch_idx():
                idx_fire(idxg_n, idxs_n, p + 1)

            g_fire(idxg_c, 1, row1, gsb)
            g_drain(idxg_c, 0, row0, gsa)
            s_fire(idxs_c, 0, row0, ssa)
            g_drain(idxg_c, 1, row1, gsb)
            s_drain(idxs_c, 0, row0, ssa)
            s_fire(idxs_c, 1, row1, ssb)

            @pl.when(p + 1 < NP)
            def _start_next():
                idx_drain(idxg_n, idxs_n, p + 1)
                g_fire(idxg_n, 0, row0, gsa)

            s_drain(idxs_c, 1, row1, ssb)

        idx_load_sync(idxga, idxsa, jnp.int32(0))
        g_fire(idxga, 0, row0, gsa)

        def body(t, _):
            stage(idxga, idxsa, idxgb, idxsb, 2 * t)
            stage(idxgb, idxsb, idxga, idxsa, 2 * t + 1)
            return _

        lax.fori_loop(0, NP // 2, body, None)

    def mul8(n, op):
        def body(i, _):
            for u in range(8):
                op(i * 8 + u)
            return _

        lax.fori_loop(0, n // 8, body, None)

    def rezero(off, n):
        base = s * LPT + off
        return [pltpu.async_copy(zbuf.at[pl.ds(0, m)],
                                 acc.at[pl.ds(base + z, m)], zsem)
                for z, m in _zero_chunks(n)]

    def dump_inv(t):
        rbase = s * LPT

        def block(off, n):
            pltpu.sync_copy(acc.at[pl.ds(rbase + off, n)],
                            row0.at[pl.ds(0, n)])
            zd = rezero(off, n)

            def op(r):
                row0[r, :] = 1.0 / jnp.maximum(row0[r, :], 1.0)

            mul8(n, op)
            pltpu.sync_copy(row0.at[pl.ds(0, n)],
                            inv.at[c].at[pl.ds(t * NUP + rbase + off, n)])
            for d_ in zd:
                d_.wait()

        stripe_blocks(block)

    def dump_pass(u2i, w2, hslot, out2, col, inv_base):
        rbase = s * LPT

        def block(off, n):
            pltpu.sync_copy(acc.at[pl.ds(rbase + off, n)],
                            row0.at[pl.ds(0, n)])
            zd = rezero(off, n)
            pltpu.sync_copy(inv.at[c].at[pl.ds(inv_base + rbase + off, n)],
                            row1.at[pl.ds(0, n)])

            def op(r):
                row0[r, :] = row0[r, :] * row1[r, :]

            mul8(n, op)

            @pl.when(w2)
            def _w2():
                pltpu.sync_copy(
                    row0.at[pl.ds(0, n)],
                    big2.at[out2].at[pl.ds(rbase + off, n),
                                     pl.ds(col * 16, 16)])

            @pl.when(u2i)
            def _also_cm():
                pltpu.sync_copy(row0.at[pl.ds(0, n)],
                                big.at[hslot].at[pl.ds(rbase + off, n)])

            for d_ in zd:
                d_.wait()

        stripe_blocks(block)

    def max_cols(a2, b2, k, clamp):
        rbase = jnp.minimum(s * LPT, clamp)
        cs = pl.ds(k * 16, 16)

        def block(off, n):
            pltpu.sync_copy(a2.at[pl.ds(rbase + off, n), cs],
                            row0.at[pl.ds(0, n)])
            pltpu.sync_copy(b2.at[pl.ds(rbase + off, n), cs],
                            row1.at[pl.ds(0, n)])

            def op(r):
                row0[r, :] = jnp.maximum(row0[r, :], row1[r, :])

            mul8(n, op)
            pltpu.sync_copy(row0.at[pl.ds(0, n)],
                            big.at[SLOT_U + k].at[pl.ds(rbase + off, n)])

        stripe_blocks(block)

    def phase_a(kk, _):
        k = 2 * c + kk
        rbase = jnp.minimum(s * LPT, CLAMP)
        cs = pl.ds(k * 16, 16)

        def block(off, n):
            pltpu.sync_copy(u0.at[pl.ds(rbase + off, n), cs],
                            row0.at[pl.ds(0, n)])
            pltpu.sync_copy(row0.at[pl.ds(0, n)],
                            big.at[SLOT_U + k].at[pl.ds(rbase + off, n)])

        stripe_blocks(block)

        @pl.when(s == 0)
        def _pad_fill():
            pltpu.sync_copy(zbuf.at[pl.ds(0, NUP - NU)],
                            big.at[SLOT_U + k].at[pl.ds(NU, NUP - NU)])
        return _

    lax.fori_loop(0, 2, phase_a, None)
    zero_acc()
    barrier()

    def phase_b(t, _):
        sdim = jnp.where(t < 2, 2 * t + 1, 2 * t - 4)
        count_sum(sdim)
        barrier()
        dump_inv(t)
        barrier()
        return _

    lax.fori_loop(0, 4, phase_b, None)

    def phase_c(kk, _):
        k = 2 * c + kk

        def layer(l, carry):
            def one_pass(p, cc):
                d = p // 2
                u2i = (p % 2) == 0
                src = jnp.where(u2i, SLOT_U + k, SLOT_H + d * 4 + k)
                gdim = jnp.where(u2i, 2 * d, 2 * d + 1)
                sdim = jnp.where(u2i, 2 * d + 1, 2 * d)
                inv_base = jnp.where(u2i, d, 2 + d) * NUP
                out2 = jnp.where(u2i, 2 + d, d)
                w2 = jnp.logical_or(jnp.logical_not(u2i), l == 1)
                seg_sum(big.at[src], gdim, sdim)
                barrier()
                dump_pass(u2i, w2, SLOT_H + d * 4 + k, out2, k, inv_base)
                barrier()
                return cc

            lax.fori_loop(0, 4, one_pass, None)

            @pl.when(l == 0)
            def _update_u():
                max_cols(big2.at[OUT_VS], big2.at[OUT_VT], k, NUP - LPT)
            barrier()
            return carry

        lax.fori_loop(0, 2, layer, None)
        return _

    lax.fori_loop(0, 2, phase_c, None)


@functools.cache
def _build():
    mesh = plsc.VectorSubcoreMesh(core_axis_name="c", subcore_axis_name="s")
    return pl.kernel(
        _sc_body,
        out_type=[
            jax.ShapeDtypeStruct((NSLOT, NUP, 16), _f32),
            jax.ShapeDtypeStruct((4, NUP, 64), _f32),
            jax.ShapeDtypeStruct((2, 4 * NUP, 16), _f32),
        ],
        mesh=mesh,
        scratch_types=[
            pltpu.VMEM((BLK_R, 16), _f32),
            pltpu.VMEM((BLK_R, 16), _f32),
            pltpu.VMEM((DBLK, 128), jnp.int32),
            pltpu.VMEM((DBLK, 128), jnp.int32),
            pltpu.VMEM((DBLK, 128), jnp.int32),
            pltpu.VMEM((DBLK, 128), jnp.int32),
            pltpu.VMEM((ZR, 16), _f32),
            pltpu.VMEM((ZR, 16), _f32),
            pltpu.VMEM_SHARED((NUP, 16), _f32),
            pltpu.SemaphoreType.DMA,
            pltpu.SemaphoreType.DMA,
            pltpu.SemaphoreType.DMA,
            pltpu.SemaphoreType.DMA,
            pltpu.SemaphoreType.DMA,
            pltpu.SemaphoreType.DMA,
        ],
        compiler_params=pltpu.CompilerParams(use_tc_tiling_on_sc=False),
    )


_NPAD = EPAD - E
_PADS = [np.int32(b) + (np.arange(_NPAD, dtype=np.int32) % DUM)
         for b in (NU, NI, NU, NI)]


def kernel(user_emb_src, user_emb_tgt, item_emb_src, item_emb_tgt,
           edge_u_src, edge_i_src, edge_u_tgt, edge_i_tgt):
    del item_emb_src, item_emb_tgt
    edges = jnp.concatenate([
        edge_u_src, _PADS[0], edge_i_src, _PADS[1],
        edge_u_tgt, _PADS[2], edge_i_tgt, _PADS[3],
    ]).reshape(4, ER, 128)
    u0 = jnp.maximum(user_emb_src, user_emb_tgt)
    _, big2, _ = _build()(edges, u0)
    return jnp.concatenate([
        big2[OUT_VS, :NU], big2[OUT_VT, :NU],
        big2[OUT_HS, :NI], big2[OUT_HT, :NI],
    ])


# speedup vs baseline: 24.6777x; 1.1662x over previous
import functools

import jax
import jax.numpy as jnp
import numpy as np
from jax import lax
from jax.experimental import pallas as pl
from jax.experimental.pallas import tpu as pltpu
from jax.experimental.pallas import tpu_sc as plsc

NU = 100000
NI = 60000
E = 600000
DUM = 64
NUP = 100096
LPT = NUP // 16
CLAMP = NU - LPT
DBLK = 8
NBLK = 76
NP = NBLK // 2
EPAD = 16 * NBLK * 512
ER = EPAD // 128
QROW = NBLK * 4
BLK_R = 512
NFULL = LPT // BLK_R
RTAIL = LPT - NFULL * BLK_R
LPT_I = 3760
NFULL_I = LPT_I // BLK_R
RTAIL_I = LPT_I - NFULL_I * BLK_R
UT15 = NU - 15 * LPT - NFULL * BLK_R
IT15 = NI - 15 * LPT_I - NFULL_I * BLK_R
OB_VS, OB_VT, OB_HS, OB_HT = 0, NU, 2 * NU, 2 * NU + NI
ZR = 128
SLOT_U, SLOT_H = 0, 4
NSLOT = 12

_f32 = jnp.float32


def _fill(buf, nrows, val):
    v = jnp.full((16,), val, _f32)

    def body(i, _):
        for u in range(4):
            buf[i * 4 + u, :] = v
        return _

    lax.fori_loop(0, nrows // 4, body, None)


def _zero_chunks(n):
    out, off = [], 0
    while off < n:
        m = min(ZR, n - off)
        out.append((off, m))
        off += m
    return out


def _sc_body(edges, u0, big, big2, inv, row0, row1, idxga, idxsa,
             idxgb, idxsb, zbuf, obuf, acc, gsa, gsb, ssa, ssb, zsem, isem):
    c = lax.axis_index("c")
    s = lax.axis_index("s")
    barrier = plsc.subcore_barrier

    _fill(zbuf, ZR, 0.0)
    _fill(obuf, ZR, 1.0)

    def stripe_blocks(fn):
        lax.fori_loop(0, NFULL, lambda t, _: (fn(t * BLK_R, BLK_R), _)[1],
                      None)
        fn(NFULL * BLK_R, RTAIL)

    def zero_acc():
        base = s * LPT

        def grp(g, _):
            off = g * (8 * ZR)
            descs = [pltpu.async_copy(zbuf,
                                      acc.at[pl.ds(base + off + j * ZR, ZR)],
                                      zsem)
                     for j in range(8)]
            for d_ in descs:
                d_.wait()
            return _

        lax.fori_loop(0, LPT // (8 * ZR), grp, None)
        tail = LPT - (LPT // (8 * ZR)) * 8 * ZR
        pltpu.async_copy(zbuf.at[pl.ds(0, tail)],
                         acc.at[pl.ds(base + LPT - tail, tail)], zsem).wait()

    def count_sum(sdim):
        def idx_fire(idxs, p):
            pltpu.async_copy(edges.at[sdim].at[pl.ds(s * QROW + p * DBLK,
                                                     DBLK)], idxs, isem)

        def idx_drain(idxs, p):
            pltpu.make_async_copy(edges.at[sdim].at[pl.ds(s * QROW + p * DBLK,
                                                          DBLK)], idxs,
                                  isem).wait()

        def stage(idxs_c, idxs_n, p):
            @pl.when(p + 1 < NP)
            def _prefetch():
                idx_fire(idxs_n, p + 1)
            for j in range(DBLK):
                pltpu.async_copy(obuf, acc.at[idxs_c.at[j]], ssa, add=True)
            for j in range(DBLK):
                pltpu.make_async_copy(obuf, acc.at[idxs_c.at[j]],
                                      ssa).wait()

            @pl.when(p + 1 < NP)
            def _ready():
                idx_drain(idxs_n, p + 1)

        pltpu.sync_copy(edges.at[sdim].at[pl.ds(s * QROW, DBLK)], idxsa)

        def cbody(t, _):
            stage(idxsa, idxsb, 2 * t)
            stage(idxsb, idxsa, 2 * t + 1)
            return _

        lax.fori_loop(0, NP // 2, cbody, None)

    def seg_sum(src, gdim, sdim):
        def idx_load_sync(idxg, idxs, p):
            rbase = s * QROW + p * DBLK
            pltpu.sync_copy(edges.at[gdim].at[pl.ds(rbase, DBLK)], idxg)
            pltpu.sync_copy(edges.at[sdim].at[pl.ds(rbase, DBLK)], idxs)

        def idx_fire(idxg, idxs, p):
            rbase = s * QROW + p * DBLK
            pltpu.async_copy(edges.at[gdim].at[pl.ds(rbase, DBLK)], idxg,
                             isem)
            pltpu.async_copy(edges.at[sdim].at[pl.ds(rbase, DBLK)], idxs,
                             isem)

        def idx_drain(idxg, idxs, p):
            rbase = s * QROW + p * DBLK
            pltpu.make_async_copy(edges.at[gdim].at[pl.ds(rbase, DBLK)],
                                  idxg, isem).wait()
            pltpu.make_async_copy(edges.at[sdim].at[pl.ds(rbase, DBLK)],
                                  idxs, isem).wait()

        def g_fire(idxg, half, buf, sem):
            for j in range(4):
                pltpu.async_copy(src.at[idxg.at[half * 4 + j]],
                                 buf.at[pl.ds(j * 128, 128)], sem)

        def g_drain(idxg, half, buf, sem):
            for j in range(4):
                pltpu.make_async_copy(src.at[idxg.at[half * 4 + j]],
                                      buf.at[pl.ds(j * 128, 128)],
                                      sem).wait()

        def s_fire(idxs, half, buf, sem):
            for j in range(4):
                pltpu.async_copy(buf.at[pl.ds(j * 128, 128)],
                                 acc.at[idxs.at[half * 4 + j]], sem,
                                 add=True)

        def s_drain(idxs, half, buf, sem):
            for j in range(4):
                pltpu.make_async_copy(buf.at[pl.ds(j * 128, 128)],
                                      acc.at[idxs.at[half * 4 + j]],
                                      sem).wait()

        def stage(idxg_c, idxs_c, idxg_n, idxs_n, p):
            @pl.when(p + 1 < NP)
            def _prefetch_idx():
                idx_fire(idxg_n, idxs_n, p + 1)

            g_fire(idxg_c, 1, row1, gsb)
            g_drain(idxg_c, 0, row0, gsa)
            s_fire(idxs_c, 0, row0, ssa)
            g_drain(idxg_c, 1, row1, gsb)
            s_drain(idxs_c, 0, row0, ssa)
            s_fire(idxs_c, 1, row1, ssb)

            @pl.when(p + 1 < NP)
            def _start_next():
                idx_drain(idxg_n, idxs_n, p + 1)
                g_fire(idxg_n, 0, row0, gsa)

            s_drain(idxs_c, 1, row1, ssb)

        idx_load_sync(idxga, idxsa, jnp.int32(0))
        g_fire(idxga, 0, row0, gsa)

        def body(t, _):
            stage(idxga, idxsa, idxgb, idxsb, 2 * t)
            stage(idxgb, idxsb, idxga, idxsa, 2 * t + 1)
            return _

        lax.fori_loop(0, NP // 2, body, None)

    def mul8(n, op):
        def body(i, _):
            for u in range(8):
                op(i * 8 + u)
            return _

        lax.fori_loop(0, n // 8, body, None)

    def rezero_at(pos, n):
        return [pltpu.async_copy(zbuf.at[pl.ds(0, m)],
                                 acc.at[pl.ds(pos + z, m)], zsem)
                for z, m in _zero_chunks(n)]

    def dump_inv(t):
        rbase = s * LPT

        def block(off, n):
            pltpu.sync_copy(acc.at[pl.ds(rbase + off, n)],
                            row0.at[pl.ds(0, n)])
            zd = rezero_at(rbase + off, n)

            def op(r):
                row0[r, :] = 1.0 / jnp.maximum(row0[r, :], 1.0)

            mul8(n, op)
            pltpu.sync_copy(row0.at[pl.ds(0, n)],
                            inv.at[c].at[pl.ds(t * NUP + rbase + off, n)])
            for d_ in zd:
                d_.wait()

        stripe_blocks(block)

    def dump_res(item, w2, hslot, ob, col, inv_base, lpt, nfull, rtail,
                 t15):
        rbase = s * lpt

        def block(off, n, tail):
            pltpu.sync_copy(acc.at[pl.ds(rbase + off, n)],
                            row0.at[pl.ds(0, n)])
            zd = rezero_at(rbase + off, n)
            pltpu.sync_copy(inv.at[c].at[pl.ds(inv_base + rbase + off, n)],
                            row1.at[pl.ds(0, n)])

            def op(r):
                row0[r, :] = row0[r, :] * row1[r, :]

            mul8(n, op)

            def w2_block(m):
                pltpu.sync_copy(
                    row0.at[pl.ds(0, m)],
                    big2.at[pl.ds(ob + rbase + off, m),
                            pl.ds(col * 16, 16)])

            @pl.when(w2)
            def _w2():
                if not tail:
                    w2_block(n)
                else:
                    @pl.when(s < 15)
                    def _full():
                        w2_block(n)

                    @pl.when(s == 15)
                    def _short():
                        w2_block(t15)

            if item:
                pltpu.sync_copy(row0.at[pl.ds(0, n)],
                                big.at[hslot].at[pl.ds(rbase + off, n)])
            for d_ in zd:
                d_.wait()

        lax.fori_loop(0, nfull,
                      lambda t, _: (block(t * BLK_R, BLK_R, False), _)[1],
                      None)
        block(nfull * BLK_R, rtail, True)

    def max_cols(abase, bbase, k):
        rbase = s * LPT
        cs = pl.ds(k * 16, 16)

        def block(off, n):
            pltpu.sync_copy(big2.at[pl.ds(abase + rbase + off, n), cs],
                            row0.at[pl.ds(0, n)])
            pltpu.sync_copy(big2.at[pl.ds(bbase + rbase + off, n), cs],
                            row1.at[pl.ds(0, n)])

            def op(r):
                row0[r, :] = jnp.maximum(row0[r, :], row1[r, :])

            mul8(n, op)
            pltpu.sync_copy(row0.at[pl.ds(0, n)],
                            big.at[SLOT_U + k].at[pl.ds(rbase + off, n)])

        stripe_blocks(block)

    def phase_a(kk, _):
        k = 2 * c + kk
        rbase = jnp.minimum(s * LPT, CLAMP)
        cs = pl.ds(k * 16, 16)

        def block(off, n):
            pltpu.sync_copy(u0.at[pl.ds(rbase + off, n), cs],
                            row0.at[pl.ds(0, n)])
            pltpu.sync_copy(row0.at[pl.ds(0, n)],
                            big.at[SLOT_U + k].at[pl.ds(rbase + off, n)])

        stripe_blocks(block)

        @pl.when(s == 0)
        def _pad_fill():
            pltpu.sync_copy(zbuf.at[pl.ds(0, NUP - NU)],
                            big.at[SLOT_U + k].at[pl.ds(NU, NUP - NU)])
        return _

    lax.fori_loop(0, 2, phase_a, None)
    zero_acc()
    barrier()

    def phase_b(t, _):
        sdim = jnp.where(t < 2, 2 * t + 1, 2 * t - 4)
        count_sum(sdim)
        barrier()
        dump_inv(t)
        barrier()
        return _

    lax.fori_loop(0, 4, phase_b, None)

    def phase_c(kk, _):
        k = 2 * c + kk

        def layer(l, carry):
            def dom(d, cc):
                hslot = SLOT_H + d * 4 + k
                seg_sum(big.at[SLOT_U + k], 2 * d, 2 * d + 1)
                barrier()
                dump_res(True, l == 1, hslot, OB_HS + d * NI, k,
                         d * NUP, LPT_I, NFULL_I, RTAIL_I, IT15)
                barrier()
                seg_sum(big.at[hslot], 2 * d + 1, 2 * d)
                barrier()
                dump_res(False, True, hslot, d * NU, k,
                         (2 + d) * NUP, LPT, NFULL, RTAIL, UT15)
                barrier()
                return cc

            lax.fori_loop(0, 2, dom, None)

            @pl.when(l == 0)
            def _update_u():
                max_cols(OB_VS, OB_VT, k)
            barrier()
            return carry

        lax.fori_loop(0, 2, layer, None)
        return _

    lax.fori_loop(0, 2, phase_c, None)


@functools.cache
def _build():
    mesh = plsc.VectorSubcoreMesh(core_axis_name="c", subcore_axis_name="s")
    return pl.kernel(
        _sc_body,
        out_type=[
            jax.ShapeDtypeStruct((NSLOT, NUP, 16), _f32),
            jax.ShapeDtypeStruct((2 * NU + 2 * NI, 64), _f32),
            jax.ShapeDtypeStruct((2, 4 * NUP, 16), _f32),
        ],
        mesh=mesh,
        scratch_types=[
            pltpu.VMEM((BLK_R, 16), _f32),
            pltpu.VMEM((BLK_R, 16), _f32),
            pltpu.VMEM((DBLK, 128), jnp.int32),
            pltpu.VMEM((DBLK, 128), jnp.int32),
            pltpu.VMEM((DBLK, 128), jnp.int32),
            pltpu.VMEM((DBLK, 128), jnp.int32),
            pltpu.VMEM((ZR, 16), _f32),
            pltpu.VMEM((ZR, 16), _f32),
            pltpu.VMEM_SHARED((NUP, 16), _f32),
            pltpu.SemaphoreType.DMA,
            pltpu.SemaphoreType.DMA,
            pltpu.SemaphoreType.DMA,
            pltpu.SemaphoreType.DMA,
            pltpu.SemaphoreType.DMA,
            pltpu.SemaphoreType.DMA,
        ],
        compiler_params=pltpu.CompilerParams(use_tc_tiling_on_sc=False),
    )


_NPAD = EPAD - E
_PADS = [np.int32(b) + (np.arange(_NPAD, dtype=np.int32) % DUM)
         for b in (NU, NI, NU, NI)]


def kernel(user_emb_src, user_emb_tgt, item_emb_src, item_emb_tgt,
           edge_u_src, edge_i_src, edge_u_tgt, edge_i_tgt):
    del item_emb_src, item_emb_tgt
    edges = jnp.concatenate([
        edge_u_src, _PADS[0], edge_i_src, _PADS[1],
        edge_u_tgt, _PADS[2], edge_i_tgt, _PADS[3],
    ]).reshape(4, ER, 128)
    u0 = jnp.maximum(user_emb_src, user_emb_tgt)
    _, out, _ = _build()(edges, u0)
    return out


# speedup vs baseline: 25.5888x; 1.0369x over previous
import functools

import jax
import jax.numpy as jnp
import numpy as np
from jax import lax
from jax.experimental import pallas as pl
from jax.experimental.pallas import tpu as pltpu
from jax.experimental.pallas import tpu_sc as plsc

NU = 100000
NI = 60000
E = 600000
DUM = 64
NUP = 100096
LPT = NUP // 16
CLAMP = NU - LPT
DBLK = 8
NBLK = 76
NP = NBLK // 2
EPAD = 16 * NBLK * 512
ER = EPAD // 128
QROW = NBLK * 4
BLK_R = 512
NFULL = LPT // BLK_R
RTAIL = LPT - NFULL * BLK_R
LPT_I = 3760
NFULL_I = LPT_I // BLK_R
RTAIL_I = LPT_I - NFULL_I * BLK_R
UT15 = NU - 15 * LPT - NFULL * BLK_R
IT15 = NI - 15 * LPT_I - NFULL_I * BLK_R
OB_VS, OB_VT, OB_HS, OB_HT = 0, NU, 2 * NU, 2 * NU + NI
ZR = 128
SLOT_U, SLOT_H = 0, 4
NSLOT = 12

_f32 = jnp.float32


def _fill(buf, nrows, val):
    v = jnp.full((16,), val, _f32)

    def body(i, _):
        for u in range(4):
            buf[i * 4 + u, :] = v
        return _

    lax.fori_loop(0, nrows // 4, body, None)


def _zero_chunks(n):
    out, off = [], 0
    while off < n:
        m = min(ZR, n - off)
        out.append((off, m))
        off += m
    return out


def _sc_body(edges, u0, big, big2, inv, row0, row1, idxga, idxsa,
             idxgb, idxsb, zbuf, obuf, acc, gsa, gsb, ssa, ssb, zsem, isem):
    c = lax.axis_index("c")
    s = lax.axis_index("s")
    barrier = plsc.subcore_barrier

    _fill(zbuf, ZR, 0.0)
    _fill(obuf, ZR, 1.0)

    def stripe_blocks(fn):
        lax.fori_loop(0, NFULL, lambda t, _: (fn(t * BLK_R, BLK_R), _)[1],
                      None)
        fn(NFULL * BLK_R, RTAIL)

    def zero_acc():
        base = s * LPT

        def grp(g, _):
            off = g * (8 * ZR)
            descs = [pltpu.async_copy(zbuf,
                                      acc.at[pl.ds(base + off + j * ZR, ZR)],
                                      zsem)
                     for j in range(8)]
            for d_ in descs:
                d_.wait()
            return _

        lax.fori_loop(0, LPT // (8 * ZR), grp, None)
        tail = LPT - (LPT // (8 * ZR)) * 8 * ZR
        pltpu.async_copy(zbuf.at[pl.ds(0, tail)],
                         acc.at[pl.ds(base + LPT - tail, tail)], zsem).wait()

    def count_sum(sdim):
        def idx_fire(idxs, p):
            pltpu.async_copy(edges.at[sdim].at[pl.ds(s * QROW + p * DBLK,
                                                     DBLK)], idxs, isem)

        def idx_drain(idxs, p):
            pltpu.make_async_copy(edges.at[sdim].at[pl.ds(s * QROW + p * DBLK,
                                                          DBLK)], idxs,
                                  isem).wait()

        def stage(idxs_c, idxs_n, p):
            @pl.when(p + 1 < NP)
            def _prefetch():
                idx_fire(idxs_n, p + 1)
            for j in range(DBLK):
                pltpu.async_copy(obuf, acc.at[idxs_c.at[j]], ssa, add=True)
            for j in range(DBLK):
                pltpu.make_async_copy(obuf, acc.at[idxs_c.at[j]],
                                      ssa).wait()

            @pl.when(p + 1 < NP)
            def _ready():
                idx_drain(idxs_n, p + 1)

        pltpu.sync_copy(edges.at[sdim].at[pl.ds(s * QROW, DBLK)], idxsa)

        def cbody(t, _):
            stage(idxsa, idxsb, 2 * t)
            stage(idxsb, idxsa, 2 * t + 1)
            return _

        lax.fori_loop(0, NP // 2, cbody, None)

    def seg_sum(src, gdim, sdim):
        def idx_load_sync(idxg, idxs, p):
            rbase = s * QROW + p * DBLK
            pltpu.sync_copy(edges.at[gdim].at[pl.ds(rbase, DBLK)], idxg)
            pltpu.sync_copy(edges.at[sdim].at[pl.ds(rbase, DBLK)], idxs)

        def idx_fire(idxg, idxs, p):
            rbase = s * QROW + p * DBLK
            pltpu.async_copy(edges.at[gdim].at[pl.ds(rbase, DBLK)], idxg,
                             isem)
            pltpu.async_copy(edges.at[sdim].at[pl.ds(rbase, DBLK)], idxs,
                             isem)

        def idx_drain(idxg, idxs, p):
            rbase = s * QROW + p * DBLK
            pltpu.make_async_copy(edges.at[gdim].at[pl.ds(rbase, DBLK)],
                                  idxg, isem).wait()
            pltpu.make_async_copy(edges.at[sdim].at[pl.ds(rbase, DBLK)],
                                  idxs, isem).wait()

        def g_fire(idxg, half, buf, sem):
            for j in range(4):
                pltpu.async_copy(src.at[idxg.at[half * 4 + j]],
                                 buf.at[pl.ds(j * 128, 128)], sem)

        def g_drain(idxg, half, buf, sem):
            for j in range(4):
                pltpu.make_async_copy(src.at[idxg.at[half * 4 + j]],
                                      buf.at[pl.ds(j * 128, 128)],
                                      sem).wait()

        def s_fire(idxs, half, buf, sem):
            for j in range(4):
                pltpu.async_copy(buf.at[pl.ds(j * 128, 128)],
                                 acc.at[idxs.at[half * 4 + j]], sem,
                                 add=True)

        def s_drain(idxs, half, buf, sem):
            for j in range(4):
                pltpu.make_async_copy(buf.at[pl.ds(j * 128, 128)],
                                      acc.at[idxs.at[half * 4 + j]],
                                      sem).wait()

        def stage(idxg_c, idxs_c, idxg_n, idxs_n, p):
            @pl.when(p + 1 < NP)
            def _prefetch_idx():
                idx_fire(idxg_n, idxs_n, p + 1)

            g_fire(idxg_c, 1, row1, gsb)
            g_drain(idxg_c, 0, row0, gsa)
            s_fire(idxs_c, 0, row0, ssa)
            g_drain(idxg_c, 1, row1, gsb)
            s_drain(idxs_c, 0, row0, ssa)
            s_fire(idxs_c, 1, row1, ssb)

            @pl.when(p + 1 < NP)
            def _start_next():
                idx_drain(idxg_n, idxs_n, p + 1)
                g_fire(idxg_n, 0, row0, gsa)

            s_drain(idxs_c, 1, row1, ssb)

        idx_load_sync(idxga, idxsa, jnp.int32(0))
        g_fire(idxga, 0, row0, gsa)

        def body(t, _):
            stage(idxga, idxsa, idxgb, idxsb, 2 * t)
            stage(idxgb, idxsb, idxga, idxsa, 2 * t + 1)
            return _

        lax.fori_loop(0, NP // 2, body, None)

    def mul8(n, op):
        def body(i, _):
            for u in range(8):
                op(i * 8 + u)
            return _

        lax.fori_loop(0, n // 8, body, None)

    def rezero_at(pos, n):
        return [pltpu.async_copy(zbuf.at[pl.ds(0, m)],
                                 acc.at[pl.ds(pos + z, m)], zsem)
                for z, m in _zero_chunks(n)]

    def dump_inv(t):
        rbase = s * LPT

        def block(off, n):
            pltpu.sync_copy(acc.at[pl.ds(rbase + off, n)],
                            row0.at[pl.ds(0, n)])
            zd = rezero_at(rbase + off, n)

            def op(r):
                row0[r, :] = 1.0 / jnp.maximum(row0[r, :], 1.0)

            mul8(n, op)
            pltpu.sync_copy(row0.at[pl.ds(0, n)],
                            inv.at[c].at[pl.ds(t * NUP + rbase + off, n)])
            for d_ in zd:
                d_.wait()

        stripe_blocks(block)

    def dump_res(item, w2, hslot, ob, col, inv_base, lpt, nfull, rtail,
                 t15):
        rbase = s * lpt

        def block(off, n, tail):
            d1 = pltpu.async_copy(acc.at[pl.ds(rbase + off, n)],
                                  row0.at[pl.ds(0, n)], gsa)
            d2 = pltpu.async_copy(
                inv.at[c].at[pl.ds(inv_base + rbase + off, n)],
                row1.at[pl.ds(0, n)], gsb)
            d1.wait()
            d2.wait()
            zd = rezero_at(rbase + off, n)

            def op(r):
                row0[r, :] = row0[r, :] * row1[r, :]

            mul8(n, op)

            def w2_block(m):
                pltpu.sync_copy(
                    row0.at[pl.ds(0, m)],
                    big2.at[pl.ds(ob + rbase + off, m),
                            pl.ds(col * 16, 16)])

            @pl.when(w2)
            def _w2():
                if not tail:
                    w2_block(n)
                else:
                    @pl.when(s < 15)
                    def _full():
                        w2_block(n)

                    @pl.when(s == 15)
                    def _short():
                        w2_block(t15)

            if item:
                pltpu.sync_copy(row0.at[pl.ds(0, n)],
                                big.at[hslot].at[pl.ds(rbase + off, n)])
            for d_ in zd:
                d_.wait()

        lax.fori_loop(0, nfull,
                      lambda t, _: (block(t * BLK_R, BLK_R, False), _)[1],
                      None)
        block(nfull * BLK_R, rtail, True)

    def max_cols(abase, bbase, k):
        rbase = s * LPT
        cs = pl.ds(k * 16, 16)

        def block(off, n):
            d1 = pltpu.async_copy(big2.at[pl.ds(abase + rbase + off, n), cs],
                                  row0.at[pl.ds(0, n)], gsa)
            d2 = pltpu.async_copy(big2.at[pl.ds(bbase + rbase + off, n), cs],
                                  row1.at[pl.ds(0, n)], gsb)
            d1.wait()
            d2.wait()

            def op(r):
                row0[r, :] = jnp.maximum(row0[r, :], row1[r, :])

            mul8(n, op)
            pltpu.sync_copy(row0.at[pl.ds(0, n)],
                            big.at[SLOT_U + k].at[pl.ds(rbase + off, n)])

        stripe_blocks(block)

    def phase_a(kk, _):
        k = 2 * c + kk
        rbase = jnp.minimum(s * LPT, CLAMP)
        cs = pl.ds(k * 16, 16)

        def block(off, n):
            pltpu.sync_copy(u0.at[pl.ds(rbase + off, n), cs],
                            row0.at[pl.ds(0, n)])
            pltpu.sync_copy(row0.at[pl.ds(0, n)],
                            big.at[SLOT_U + k].at[pl.ds(rbase + off, n)])

        stripe_blocks(block)

        @pl.when(s == 0)
        def _pad_fill():
            pltpu.sync_copy(zbuf.at[pl.ds(0, NUP - NU)],
                            big.at[SLOT_U + k].at[pl.ds(NU, NUP - NU)])
        return _

    lax.fori_loop(0, 2, phase_a, None)
    zero_acc()
    barrier()

    def phase_b(t, _):
        sdim = jnp.where(t < 2, 2 * t + 1, 2 * t - 4)
        count_sum(sdim)
        barrier()
        dump_inv(t)
        barrier()
        return _

    lax.fori_loop(0, 4, phase_b, None)

    def phase_c(kk, _):
        k = 2 * c + kk

        def layer(l, carry):
            def dom(d, cc):
                hslot = SLOT_H + d * 4 + k
                seg_sum(big.at[SLOT_U + k], 2 * d, 2 * d + 1)
                barrier()
                dump_res(True, l == 1, hslot, OB_HS + d * NI, k,
                         d * NUP, LPT_I, NFULL_I, RTAIL_I, IT15)
                barrier()
                seg_sum(big.at[hslot], 2 * d + 1, 2 * d)
                barrier()
                dump_res(False, True, hslot, d * NU, k,
                         (2 + d) * NUP, LPT, NFULL, RTAIL, UT15)
                barrier()
                return cc

            lax.fori_loop(0, 2, dom, None)

            @pl.when(l == 0)
            def _update_u():
                max_cols(OB_VS, OB_VT, k)
            barrier()
            return carry

        lax.fori_loop(0, 2, layer, None)
        return _

    lax.fori_loop(0, 2, phase_c, None)


@functools.cache
def _build():
    mesh = plsc.VectorSubcoreMesh(core_axis_name="c", subcore_axis_name="s")
    return pl.kernel(
        _sc_body,
        out_type=[
            jax.ShapeDtypeStruct((NSLOT, NUP, 16), _f32),
            jax.ShapeDtypeStruct((2 * NU + 2 * NI, 64), _f32),
            jax.ShapeDtypeStruct((2, 4 * NUP, 16), _f32),
        ],
        mesh=mesh,
        scratch_types=[
            pltpu.VMEM((BLK_R, 16), _f32),
            pltpu.VMEM((BLK_R, 16), _f32),
            pltpu.VMEM((DBLK, 128), jnp.int32),
            pltpu.VMEM((DBLK, 128), jnp.int32),
            pltpu.VMEM((DBLK, 128), jnp.int32),
            pltpu.VMEM((DBLK, 128), jnp.int32),
            pltpu.VMEM((ZR, 16), _f32),
            pltpu.VMEM((ZR, 16), _f32),
            pltpu.VMEM_SHARED((NUP, 16), _f32),
            pltpu.SemaphoreType.DMA,
            pltpu.SemaphoreType.DMA,
            pltpu.SemaphoreType.DMA,
            pltpu.SemaphoreType.DMA,
            pltpu.SemaphoreType.DMA,
            pltpu.SemaphoreType.DMA,
        ],
        compiler_params=pltpu.CompilerParams(use_tc_tiling_on_sc=False),
    )


_NPAD = EPAD - E
_PADS = [np.int32(b) + (np.arange(_NPAD, dtype=np.int32) % DUM)
         for b in (NU, NI, NU, NI)]


def kernel(user_emb_src, user_emb_tgt, item_emb_src, item_emb_tgt,
           edge_u_src, edge_i_src, edge_u_tgt, edge_i_tgt):
    del item_emb_src, item_emb_tgt
    edges = jnp.concatenate([
        edge_u_src, _PADS[0], edge_i_src, _PADS[1],
        edge_u_tgt, _PADS[2], edge_i_tgt, _PADS[3],
    ]).reshape(4, ER, 128)
    u0 = jnp.maximum(user_emb_src, user_emb_tgt)
    _, out, _ = _build()(edges, u0)
    return out
